# baseline (speedup 1.0000x reference)
import functools

import jax
import jax.numpy as jnp
from jax import lax
from jax.experimental import pallas as pl
from jax.experimental.pallas import tpu as pltpu
from jax.experimental.pallas import tpu_sc as plsc


_NC = 2
_NS = 16
_NW = _NC * _NS
_CHUNK = 128
_NBUF = 3


def kernel(inputs):
    n, c = inputs.shape
    rows_per_w = n // _NW
    nchunks = rows_per_w // _CHUNK
    mesh = plsc.VectorSubcoreMesh(core_axis_name="c", subcore_axis_name="s")

    @functools.partial(
        pl.kernel,
        mesh=mesh,
        out_type=[
            jax.ShapeDtypeStruct((n, c), jnp.float32),
            jax.ShapeDtypeStruct((n,), jnp.float32),
        ],
        scratch_types=[
            pltpu.VMEM((_NBUF, _CHUNK, c), jnp.float32),
            pltpu.VMEM((rows_per_w,), jnp.float32),
            pltpu.SemaphoreType.DMA((_NBUF,)),
            pltpu.SemaphoreType.DMA((_NBUF,)),
            pltpu.SemaphoreType.DMA,
        ],
    )
    def _sc_copy(x_hbm, out_hbm, ld_hbm, buf, zbuf, in_sems, out_sems, zsem):
        wid = lax.axis_index("s") * _NC + lax.axis_index("c")
        base = wid * rows_per_w

        def _in_copy(i):
            return pltpu.make_async_copy(
                x_hbm.at[pl.ds(base + i * _CHUNK, _CHUNK)],
                buf.at[i % _NBUF],
                in_sems.at[i % _NBUF],
            )

        def _out_copy(i):
            return pltpu.make_async_copy(
                buf.at[i % _NBUF],
                out_hbm.at[pl.ds(base + i * _CHUNK, _CHUNK)],
                out_sems.at[i % _NBUF],
            )

        for i in range(_NBUF):
            _in_copy(i).start()

        def _zfill(i, carry):
            zbuf[pl.ds(i * 16, 16)] = jnp.zeros((16,), jnp.float32)
            return carry

        lax.fori_loop(0, rows_per_w // 16, _zfill, 0)
        zcopy = pltpu.make_async_copy(
            zbuf, ld_hbm.at[pl.ds(base, rows_per_w)], zsem
        )
        zcopy.start()

        for i in range(nchunks):
            _in_copy(i).wait()
            _out_copy(i).start()
            if i + _NBUF < nchunks:
                _out_copy(i).wait()
                _in_copy(i + _NBUF).start()

        for i in range(nchunks - _NBUF, nchunks):
            _out_copy(i).wait()
        zcopy.wait()

    outputs, ld = _sc_copy(inputs)
    return (outputs, ld)
